# baseline (speedup 1.0000x reference)
import functools

import jax
import jax.numpy as jnp
from jax import lax
from jax.experimental import pallas as pl
from jax.experimental.pallas import tpu as pltpu
from jax.experimental.pallas import tpu_sc as plsc

_E = 64
_K = 2
_CAP = 128
_T = 2048
_D = 1024
_F = 1024
_NROWS = _E * _CAP
_INV_PAD = _NROWS + 256
_SENT = _T
_ZPAD = 256
_CH = 512

_NC = 2
_NS = 16
_NW = _NC * _NS
_ROWS_W = _NROWS // _NW
_TOK_W = _T // _NW


def _routing_body(logits_ref, se_ref, so_ref, ge_ref, go_ref, wbe_ref, wbo_ref):
    l = logits_ref[...]
    m = jnp.max(l, axis=1, keepdims=True)
    e = jnp.exp(l - m)
    z = jnp.sum(e, axis=1, keepdims=True)
    p = e / z
    col = lax.broadcasted_iota(jnp.int32, (_T, _E), 1)
    m1 = jnp.max(p, axis=1, keepdims=True)
    a1 = jnp.min(jnp.where(p == m1, col, _E), axis=1, keepdims=True)
    p2 = jnp.where(col == a1, -1.0, p)
    m2 = jnp.max(p2, axis=1, keepdims=True)
    a2 = jnp.min(jnp.where(p2 == m2, col, _E), axis=1, keepdims=True)
    denom = m1 + m2
    w1 = (m1 / denom)[:, 0]
    w2 = (m2 / denom)[:, 0]

    oh_e = (col == a1).astype(jnp.float32)
    oh_o = (col == a2).astype(jnp.float32)
    comb = oh_e + oh_o
    r_i = lax.broadcasted_iota(jnp.int32, (_CH, _CH), 0)
    c_i = lax.broadcasted_iota(jnp.int32, (_CH, _CH), 1)
    tl = (c_i < r_i).astype(jnp.float32)

    carry = jnp.zeros((1, _E), jnp.float32)
    pe_parts, po_parts = [], []
    for c in range(_T // _CH):
        s0 = c * _CH
        cc = comb[s0:s0 + _CH]
        b = jnp.dot(tl, cc, preferred_element_type=jnp.float32) + carry
        pe_parts.append(jnp.sum(oh_e[s0:s0 + _CH] * b, axis=1))
        po_parts.append(jnp.sum(oh_o[s0:s0 + _CH] * b, axis=1))
        carry = carry + jnp.sum(cc, axis=0, keepdims=True)
    pe = jnp.concatenate(pe_parts).astype(jnp.int32)
    po = jnp.concatenate(po_parts).astype(jnp.int32)

    id1 = a1[:, 0]
    id2 = a2[:, 0]
    ve = pe < _CAP
    vo = po < _CAP
    slot_e = id1 * _CAP + pe
    slot_o = id2 * _CAP + po
    se_ref[...] = jnp.where(ve, slot_e, _NROWS)
    so_ref[...] = jnp.where(vo, slot_o, _NROWS)
    ge_ref[...] = jnp.where(ve, slot_e, 0)
    go_ref[...] = jnp.where(vo, slot_o, 0)
    wbe_ref[...] = jnp.broadcast_to(
        jnp.where(ve, w1, 0.0)[:, None], (_T, 16))
    wbo_ref[...] = jnp.broadcast_to(
        jnp.where(vo, w2, 0.0)[:, None], (_T, 16))


_routing = pl.pallas_call(
    _routing_body,
    out_shape=[
        jax.ShapeDtypeStruct((_T,), jnp.int32),
        jax.ShapeDtypeStruct((_T,), jnp.int32),
        jax.ShapeDtypeStruct((_T,), jnp.int32),
        jax.ShapeDtypeStruct((_T,), jnp.int32),
        jax.ShapeDtypeStruct((_T, 16), jnp.float32),
        jax.ShapeDtypeStruct((_T, 16), jnp.float32),
    ],
)


_DCH = 32
_NDCH = _ROWS_W // _DCH


def _dispatch_body(xpad_hbm, se_hbm, so_hbm, xb_hbm,
                   se_v, so_v, idx_v, rows0, rows1,
                   gsem0, gsem1, wsem0, wsem1):
    wid = lax.axis_index("s") * _NC + lax.axis_index("c")
    base = wid * _ROWS_W
    gbase = base
    pltpu.sync_copy(se_hbm, se_v)
    pltpu.sync_copy(so_hbm, so_v)

    def memset_body(i, carry):
        lane = lax.iota(jnp.int32, 16) + (gbase + i * 16)
        idx_v[pl.ds(i * 16, 16)] = (lane & (_ZPAD - 1)) + _T
        return carry
    lax.fori_loop(0, _ROWS_W // 16, memset_body, 0)

    def scat_body(i, carry):
        tok = lax.iota(jnp.int32, 16) + i * 16
        for sv in (se_v, so_v):
            slot = sv[pl.ds(i * 16, 16)]
            loc = slot - gbase
            m = (loc >= 0) & (loc < _ROWS_W)
            plsc.store_scatter(idx_v, [jnp.where(m, loc, 0)], tok, mask=m)
        return carry
    lax.fori_loop(0, _T // 16, scat_body, 0)

    bufs = (rows0, rows1)
    gsems = (gsem0, gsem1)
    wsems = (wsem0, wsem1)
    copies = [None, None]
    writes = [None, None]
    for c in range(_NDCH + 1):
        if c < _NDCH:
            s = c % 2
            if writes[s] is not None:
                writes[s].wait()
            copies[s] = pltpu.async_copy(
                xpad_hbm.at[idx_v.at[pl.ds(c * _DCH, _DCH)]], bufs[s], gsems[s])
        if c > 0:
            s = (c - 1) % 2
            copies[s].wait()
            writes[s] = pltpu.async_copy(
                bufs[s], xb_hbm.at[pl.ds(base + (c - 1) * _DCH, _DCH)], wsems[s])
    writes[(_NDCH - 1) % 2].wait()
    writes[_NDCH % 2].wait()


@functools.cache
def _get_dispatch():
    return pl.kernel(
        _dispatch_body,
        out_type=jax.ShapeDtypeStruct((_NROWS, _D), jnp.float32),
        mesh=plsc.VectorSubcoreMesh(core_axis_name="c", subcore_axis_name="s",
                                    num_cores=_NC, num_subcores=_NS),
        compiler_params=pltpu.CompilerParams(needs_layout_passes=False),
        scratch_types=[
            pltpu.VMEM((_T,), jnp.int32),
            pltpu.VMEM((_T,), jnp.int32),
            pltpu.VMEM((_ROWS_W,), jnp.int32),
            pltpu.VMEM((_DCH, _D), jnp.float32),
            pltpu.VMEM((_DCH, _D), jnp.float32),
            pltpu.SemaphoreType.DMA,
            pltpu.SemaphoreType.DMA,
            pltpu.SemaphoreType.DMA,
            pltpu.SemaphoreType.DMA,
        ],
    )


def _ffn_a_body(xb_ref, wg_ref, wu_ref, wd_ref, y_ref):
    x = xb_ref[...]
    g = jnp.dot(x, wg_ref[0], preferred_element_type=jnp.float32)
    u = jnp.dot(x, wu_ref[0], preferred_element_type=jnp.float32)
    h = g * (1.0 / (1.0 + jnp.exp(-g))) * u
    y_ref[...] = jnp.dot(h, wd_ref[0], preferred_element_type=jnp.float32)


_ffn = pl.pallas_call(
    _ffn_a_body,
    grid=(_E,),
    in_specs=[
        pl.BlockSpec((_CAP, _D), lambda e: (e, 0)),
        pl.BlockSpec((1, _D, _F), lambda e: (e, 0, 0)),
        pl.BlockSpec((1, _D, _F), lambda e: (e, 0, 0)),
        pl.BlockSpec((1, _F, _D), lambda e: (e, 0, 0)),
    ],
    out_specs=pl.BlockSpec((_CAP, _D), lambda e: (e, 0)),
    out_shape=jax.ShapeDtypeStruct((_NROWS, _D), jnp.float32),
)


_CCH = 16
_NCCH = _TOK_W // _CCH


def _combine_body(y_hbm, ge_hbm, go_hbm, wbe_hbm, wbo_hbm, out_hbm,
                  ge_v, go_v, gi_v, we_v, wo_v,
                  yb0, yb1, ob0, ob1, gsem0, gsem1, wsem0, wsem1):
    wid = lax.axis_index("s") * _NC + lax.axis_index("c")
    tb = wid * _TOK_W
    pltpu.sync_copy(ge_hbm.at[pl.ds(tb, _TOK_W)], ge_v)
    pltpu.sync_copy(go_hbm.at[pl.ds(tb, _TOK_W)], go_v)
    for i in range(_TOK_W // 16):
        pos = (lax.iota(jnp.int32, 16) + i * 16) * 2
        plsc.store_scatter(gi_v, [pos], ge_v[pl.ds(i * 16, 16)])
        plsc.store_scatter(gi_v, [pos + 1], go_v[pl.ds(i * 16, 16)])
    pltpu.sync_copy(wbe_hbm.at[pl.ds(tb, _TOK_W)], we_v)
    pltpu.sync_copy(wbo_hbm.at[pl.ds(tb, _TOK_W)], wo_v)
    ybufs = (yb0, yb1)
    obufs = (ob0, ob1)
    gsems = (gsem0, gsem1)
    wsems = (wsem0, wsem1)
    gc = [None, None]
    wr = [None, None]
    for c in range(_NCCH + 1):
        if c < _NCCH:
            s = c % 2
            gc[s] = pltpu.async_copy(
                y_hbm.at[gi_v.at[pl.ds(c * 2 * _CCH, 2 * _CCH)]],
                ybufs[s], gsems[s])
        if c > 0:
            s = (c - 1) % 2
            gc[s].wait()
            if wr[s] is not None:
                wr[s].wait()
            y_b, o_b = ybufs[s], obufs[s]
            w_off = (c - 1) * _CCH

            def row_body(r, carry):
                we = we_v[w_off + r, :]
                wo = wo_v[w_off + r, :]
                for g in range(_D // 16):
                    sl = pl.ds(g * 16, 16)
                    o_b[r, sl] = we * y_b[2 * r, sl] + wo * y_b[2 * r + 1, sl]
                return carry
            lax.fori_loop(0, _CCH, row_body, 0)
            wr[s] = pltpu.async_copy(
                o_b, out_hbm.at[pl.ds(tb + w_off, _CCH)], wsems[s])
    wr[(_NCCH - 1) % 2].wait()
    wr[_NCCH % 2].wait()


@functools.cache
def _get_combine():
    return pl.kernel(
        _combine_body,
        out_type=jax.ShapeDtypeStruct((_T, _D), jnp.float32),
        mesh=plsc.VectorSubcoreMesh(core_axis_name="c", subcore_axis_name="s",
                                    num_cores=_NC, num_subcores=_NS),
        compiler_params=pltpu.CompilerParams(needs_layout_passes=False),
        scratch_types=[
            pltpu.VMEM((_TOK_W,), jnp.int32),
            pltpu.VMEM((_TOK_W,), jnp.int32),
            pltpu.VMEM((2 * _TOK_W,), jnp.int32),
            pltpu.VMEM((_TOK_W, 16), jnp.float32),
            pltpu.VMEM((_TOK_W, 16), jnp.float32),
            pltpu.VMEM((2 * _CCH, _D), jnp.float32),
            pltpu.VMEM((2 * _CCH, _D), jnp.float32),
            pltpu.VMEM((_CCH, _D), jnp.float32),
            pltpu.VMEM((_CCH, _D), jnp.float32),
            pltpu.SemaphoreType.DMA,
            pltpu.SemaphoreType.DMA,
            pltpu.SemaphoreType.DMA,
            pltpu.SemaphoreType.DMA,
        ],
    )


def kernel(hidden_states, router_logits, w_gate, w_up, w_down):
    xpad = jnp.concatenate(
        [hidden_states, jnp.zeros((_ZPAD, _D), jnp.float32)], axis=0)
    se, so, ge, go, wbe, wbo = _routing(router_logits)
    xb = _get_dispatch()(xpad, se, so)
    y = _ffn(xb, w_gate, w_up, w_down)
    return _get_combine()(y, ge, go, wbe, wbo)


# speedup vs baseline: 1.0508x; 1.0508x over previous
import functools

import jax
import jax.numpy as jnp
from jax import lax
from jax.experimental import pallas as pl
from jax.experimental.pallas import tpu as pltpu
from jax.experimental.pallas import tpu_sc as plsc

_E = 64
_K = 2
_CAP = 128
_T = 2048
_D = 1024
_F = 1024
_NROWS = _E * _CAP
_INV_PAD = _NROWS + 256
_SENT = _T
_ZPAD = 256
_CH = 512

_NC = 2
_NS = 16
_NW = _NC * _NS
_ROWS_W = _NROWS // _NW
_TOK_W = _T // _NW


def _routing_body(logits_ref, se_ref, so_ref, ge_ref, go_ref, wbe_ref, wbo_ref):
    l = logits_ref[...]
    m = jnp.max(l, axis=1, keepdims=True)
    e = jnp.exp(l - m)
    z = jnp.sum(e, axis=1, keepdims=True)
    p = e / z
    col = lax.broadcasted_iota(jnp.int32, (_T, _E), 1)
    m1 = jnp.max(p, axis=1, keepdims=True)
    a1 = jnp.min(jnp.where(p == m1, col, _E), axis=1, keepdims=True)
    p2 = jnp.where(col == a1, -1.0, p)
    m2 = jnp.max(p2, axis=1, keepdims=True)
    a2 = jnp.min(jnp.where(p2 == m2, col, _E), axis=1, keepdims=True)
    denom = m1 + m2
    w1 = (m1 / denom)[:, 0]
    w2 = (m2 / denom)[:, 0]

    oh_e = (col == a1).astype(jnp.float32)
    oh_o = (col == a2).astype(jnp.float32)
    comb = oh_e + oh_o
    r_i = lax.broadcasted_iota(jnp.int32, (_CH, _CH), 0)
    c_i = lax.broadcasted_iota(jnp.int32, (_CH, _CH), 1)
    tl = (c_i < r_i).astype(jnp.float32)

    carry = jnp.zeros((1, _E), jnp.float32)
    pe_parts, po_parts = [], []
    for c in range(_T // _CH):
        s0 = c * _CH
        cc = comb[s0:s0 + _CH]
        b = jnp.dot(tl, cc, preferred_element_type=jnp.float32) + carry
        pe_parts.append(jnp.sum(oh_e[s0:s0 + _CH] * b, axis=1))
        po_parts.append(jnp.sum(oh_o[s0:s0 + _CH] * b, axis=1))
        carry = carry + jnp.sum(cc, axis=0, keepdims=True)
    pe = jnp.concatenate(pe_parts).astype(jnp.int32)
    po = jnp.concatenate(po_parts).astype(jnp.int32)

    id1 = a1[:, 0]
    id2 = a2[:, 0]
    ve = pe < _CAP
    vo = po < _CAP
    slot_e = id1 * _CAP + pe
    slot_o = id2 * _CAP + po
    se_ref[...] = jnp.where(ve, slot_e, _NROWS)
    so_ref[...] = jnp.where(vo, slot_o, _NROWS)
    ge_ref[...] = jnp.where(ve, slot_e, 0)
    go_ref[...] = jnp.where(vo, slot_o, 0)
    wbe_ref[...] = jnp.broadcast_to(
        jnp.where(ve, w1, 0.0)[:, None], (_T, 16))
    wbo_ref[...] = jnp.broadcast_to(
        jnp.where(vo, w2, 0.0)[:, None], (_T, 16))


_routing = pl.pallas_call(
    _routing_body,
    out_shape=[
        jax.ShapeDtypeStruct((_T,), jnp.int32),
        jax.ShapeDtypeStruct((_T,), jnp.int32),
        jax.ShapeDtypeStruct((_T,), jnp.int32),
        jax.ShapeDtypeStruct((_T,), jnp.int32),
        jax.ShapeDtypeStruct((_T, 16), jnp.float32),
        jax.ShapeDtypeStruct((_T, 16), jnp.float32),
    ],
)


_DCH = 32
_NDCH = _ROWS_W // _DCH


def _dispatch_body(xpad_hbm, se_hbm, so_hbm, xb_hbm,
                   se_v, so_v, idx_v, rows0, rows1,
                   gsem0, gsem1, wsem0, wsem1):
    wid = lax.axis_index("s") * _NC + lax.axis_index("c")
    base = wid * _ROWS_W
    gbase = base
    pltpu.sync_copy(se_hbm, se_v)
    pltpu.sync_copy(so_hbm, so_v)

    def memset_body(i, carry):
        lane = lax.iota(jnp.int32, 16) + (gbase + i * 16)
        idx_v[pl.ds(i * 16, 16)] = (lane & (_ZPAD - 1)) + _T
        return carry
    lax.fori_loop(0, _ROWS_W // 16, memset_body, 0)

    def scat_body(i, carry):
        tok = lax.iota(jnp.int32, 16) + i * 16
        for sv in (se_v, so_v):
            slot = sv[pl.ds(i * 16, 16)]
            loc = slot - gbase
            m = (loc >= 0) & (loc < _ROWS_W)
            plsc.store_scatter(idx_v, [jnp.where(m, loc, 0)], tok, mask=m)
        return carry
    lax.fori_loop(0, _T // 16, scat_body, 0)

    bufs = (rows0, rows1)
    gsems = (gsem0, gsem1)
    wsems = (wsem0, wsem1)
    copies = [None, None]
    writes = [None, None]
    for c in range(_NDCH + 1):
        if c < _NDCH:
            s = c % 2
            if writes[s] is not None:
                writes[s].wait()
            copies[s] = pltpu.async_copy(
                xpad_hbm.at[idx_v.at[pl.ds(c * _DCH, _DCH)]], bufs[s], gsems[s])
        if c > 0:
            s = (c - 1) % 2
            copies[s].wait()
            writes[s] = pltpu.async_copy(
                bufs[s], xb_hbm.at[pl.ds(base + (c - 1) * _DCH, _DCH)], wsems[s])
    writes[(_NDCH - 1) % 2].wait()
    writes[_NDCH % 2].wait()


@functools.cache
def _get_dispatch():
    return pl.kernel(
        _dispatch_body,
        out_type=jax.ShapeDtypeStruct((_NROWS, _D), jnp.float32),
        mesh=plsc.VectorSubcoreMesh(core_axis_name="c", subcore_axis_name="s",
                                    num_cores=_NC, num_subcores=_NS),
        compiler_params=pltpu.CompilerParams(needs_layout_passes=False),
        scratch_types=[
            pltpu.VMEM((_T,), jnp.int32),
            pltpu.VMEM((_T,), jnp.int32),
            pltpu.VMEM((_ROWS_W,), jnp.int32),
            pltpu.VMEM((_DCH, _D), jnp.float32),
            pltpu.VMEM((_DCH, _D), jnp.float32),
            pltpu.SemaphoreType.DMA,
            pltpu.SemaphoreType.DMA,
            pltpu.SemaphoreType.DMA,
            pltpu.SemaphoreType.DMA,
        ],
    )


def _ffn_a_body(xb_ref, wg_ref, wu_ref, wd_ref, y_ref):
    x = xb_ref[...]
    g = jnp.dot(x, wg_ref[0], preferred_element_type=jnp.float32)
    u = jnp.dot(x, wu_ref[0], preferred_element_type=jnp.float32)
    h = g * (1.0 / (1.0 + jnp.exp(-g))) * u
    y_ref[...] = jnp.dot(h, wd_ref[0], preferred_element_type=jnp.float32)


_ffn = pl.pallas_call(
    _ffn_a_body,
    grid=(_E,),
    in_specs=[
        pl.BlockSpec((_CAP, _D), lambda e: (e, 0)),
        pl.BlockSpec((1, _D, _F), lambda e: (e, 0, 0)),
        pl.BlockSpec((1, _D, _F), lambda e: (e, 0, 0)),
        pl.BlockSpec((1, _F, _D), lambda e: (e, 0, 0)),
    ],
    out_specs=pl.BlockSpec((_CAP, _D), lambda e: (e, 0)),
    out_shape=jax.ShapeDtypeStruct((_NROWS, _D), jnp.float32),
)


_CCH = 16
_NCCH = _TOK_W // _CCH


def _combine_body(y_hbm, ge_hbm, go_hbm, wbe_hbm, wbo_hbm, out_hbm,
                  ge_v, go_v, we_v, wo_v,
                  ye0, ye1, yo0, yo1, gsem0, gsem1, wsem0, wsem1):
    wid = lax.axis_index("s") * _NC + lax.axis_index("c")
    tb = wid * _TOK_W
    pltpu.sync_copy(ge_hbm.at[pl.ds(tb, _TOK_W)], ge_v)
    pltpu.sync_copy(go_hbm.at[pl.ds(tb, _TOK_W)], go_v)
    pltpu.sync_copy(wbe_hbm.at[pl.ds(tb, _TOK_W)], we_v)
    pltpu.sync_copy(wbo_hbm.at[pl.ds(tb, _TOK_W)], wo_v)
    yes = (ye0, ye1)
    yos = (yo0, yo1)
    gsems = (gsem0, gsem1)
    wsems = (wsem0, wsem1)
    gec = [None, None]
    goc = [None, None]
    wr = [None, None]
    for c in range(_NCCH + 1):
        if c < _NCCH:
            s = c % 2
            if wr[s] is not None:
                wr[s].wait()
            gec[s] = pltpu.async_copy(
                y_hbm.at[ge_v.at[pl.ds(c * _CCH, _CCH)]], yes[s], gsems[s])
            goc[s] = pltpu.async_copy(
                y_hbm.at[go_v.at[pl.ds(c * _CCH, _CCH)]], yos[s], gsems[s])
        if c > 0:
            s = (c - 1) % 2
            gec[s].wait()
            goc[s].wait()
            ye_b, yo_b = yes[s], yos[s]
            w_off = (c - 1) * _CCH

            def row_body(r, carry):
                we = we_v[w_off + r, :]
                wo = wo_v[w_off + r, :]
                for g in range(_D // 16):
                    sl = pl.ds(g * 16, 16)
                    ye_b[r, sl] = we * ye_b[r, sl] + wo * yo_b[r, sl]
                return carry
            lax.fori_loop(0, _CCH, row_body, 0)
            wr[s] = pltpu.async_copy(
                ye_b, out_hbm.at[pl.ds(tb + w_off, _CCH)], wsems[s])
    wr[(_NCCH - 1) % 2].wait()
    wr[_NCCH % 2].wait()


@functools.cache
def _get_combine():
    return pl.kernel(
        _combine_body,
        out_type=jax.ShapeDtypeStruct((_T, _D), jnp.float32),
        mesh=plsc.VectorSubcoreMesh(core_axis_name="c", subcore_axis_name="s",
                                    num_cores=_NC, num_subcores=_NS),
        compiler_params=pltpu.CompilerParams(needs_layout_passes=False),
        scratch_types=[
            pltpu.VMEM((_TOK_W,), jnp.int32),
            pltpu.VMEM((_TOK_W,), jnp.int32),
            pltpu.VMEM((_TOK_W, 16), jnp.float32),
            pltpu.VMEM((_TOK_W, 16), jnp.float32),
            pltpu.VMEM((_CCH, _D), jnp.float32),
            pltpu.VMEM((_CCH, _D), jnp.float32),
            pltpu.VMEM((_CCH, _D), jnp.float32),
            pltpu.VMEM((_CCH, _D), jnp.float32),
            pltpu.SemaphoreType.DMA,
            pltpu.SemaphoreType.DMA,
            pltpu.SemaphoreType.DMA,
            pltpu.SemaphoreType.DMA,
        ],
    )


def kernel(hidden_states, router_logits, w_gate, w_up, w_down):
    xpad = jnp.concatenate(
        [hidden_states, jnp.zeros((_ZPAD, _D), jnp.float32)], axis=0)
    se, so, ge, go, wbe, wbo = _routing(router_logits)
    xb = _get_dispatch()(xpad, se, so)
    y = _ffn(xb, w_gate, w_up, w_down)
    return _get_combine()(y, ge, go, wbe, wbo)
